# baseline (speedup 1.0000x reference)
import functools

import jax
import jax.numpy as jnp
from jax import lax
from jax.experimental import pallas as pl
from jax.experimental.pallas import tpu as pltpu
from jax.experimental.pallas import tpu_sc as plsc

_K = 5
_CHUNK = 2048
_NC = 2
_NS = 16
_NW = _NC * _NS


def _topk_body(x_ref, idx_ref):
    B, D = x_ref.shape
    nch = D // _CHUNK
    sels = []
    for k in range(_K):
        best_v = jnp.full((B, 1), -jnp.inf, dtype=jnp.float32)
        best_i = jnp.zeros((B, 1), dtype=jnp.int32)
        for c in range(nch):
            v = x_ref[:, c * _CHUNK:(c + 1) * _CHUNK]
            col = jax.lax.broadcasted_iota(jnp.int32, (B, _CHUNK), 1) + c * _CHUNK
            for j in range(k):
                v = jnp.where(col == sels[j], -jnp.inf, v)
            cm = jnp.max(v, axis=1, keepdims=True)
            ci = jnp.min(jnp.where(v == cm, col, D), axis=1, keepdims=True)
            upd = cm > best_v
            best_v = jnp.where(upd, cm, best_v)
            best_i = jnp.where(upd, ci, best_i)
        sels.append(best_i)
    idx_ref[...] = jnp.concatenate(sels, axis=1)


def _sc_fill_body(D, ppw, idx_hbm, zsrc_hbm, out_hbm, zbuf, idxbuf, obuf, sem_z, sem_o):
    wid = lax.axis_index("s") * _NC + lax.axis_index("c")
    base_plane = wid * ppw

    pltpu.sync_copy(zsrc_hbm, zbuf)

    handles = []
    for j in range(ppw):
        dst = out_hbm.at[pl.ds((base_plane + j) * D, D)]
        handles.append(pltpu.async_copy(zbuf, dst, sem_z))

    abase = (base_plane // 8) * 8
    pltpu.sync_copy(idx_hbm.at[pl.ds(abase, 32)], idxbuf)
    off = base_plane - abase

    for h in handles:
        h.wait()

    iota = lax.iota(jnp.int32, 16)
    c0 = idxbuf[pl.ds(0, 16)]
    c1 = idxbuf[pl.ds(16, 16)]
    ones_handles = []
    for j in range(ppw):
        l = off + j
        pos = jnp.maximum(
            jnp.max(jnp.where(iota == l, c0, -1)),
            jnp.max(jnp.where(iota == l - 16, c1, -1)),
        )
        seg = (pos // 16) * 16
        lane = pos - seg
        obuf[pl.ds(16 * j, 16)] = jnp.where(iota == lane, 1.0, 0.0).astype(jnp.float32)
        dst = out_hbm.at[pl.ds((base_plane + j) * D + seg, 16)]
        ones_handles.append(pltpu.async_copy(obuf.at[pl.ds(16 * j, 16)], dst, sem_o))
    for h in ones_handles:
        h.wait()


def kernel(similarities):
    B, D = similarities.shape
    idx = pl.pallas_call(
        _topk_body,
        out_shape=jax.ShapeDtypeStruct((B, _K), jnp.int32),
    )(similarities)

    nplanes = B * _K
    ppw = nplanes // _NW
    idx_pad = jnp.concatenate(
        [idx.reshape(-1), jnp.zeros((64,), jnp.int32)])
    zsrc = jnp.zeros((D,), jnp.float32)

    sc_fill = functools.partial(
        pl.kernel,
        out_type=jax.ShapeDtypeStruct((nplanes * D,), jnp.float32),
        mesh=plsc.VectorSubcoreMesh(core_axis_name="c", subcore_axis_name="s"),
        compiler_params=pltpu.CompilerParams(needs_layout_passes=False),
        scratch_types=[
            pltpu.VMEM((D,), jnp.float32),
            pltpu.VMEM((32,), jnp.int32),
            pltpu.VMEM((16 * ppw,), jnp.float32),
            pltpu.SemaphoreType.DMA,
            pltpu.SemaphoreType.DMA,
        ],
    )(functools.partial(_sc_fill_body, D, ppw))

    out_flat = sc_fill(idx_pad, zsrc)
    return idx, out_flat.reshape(B, _K, D)


# speedup vs baseline: 1.4315x; 1.4315x over previous
import functools

import jax
import jax.numpy as jnp
from jax import lax
from jax.experimental import pallas as pl
from jax.experimental.pallas import tpu as pltpu
from jax.experimental.pallas import tpu_sc as plsc

_K = 5
_CHUNK = 2048
_NC = 2
_NS = 16
_NW = _NC * _NS


def _topk_body(x_ref, idx_ref):
    B, D = x_ref.shape
    nch = D // _CHUNK
    sels = []
    for k in range(_K):
        best_v = jnp.full((B, 1), -jnp.inf, dtype=jnp.float32)
        best_i = jnp.zeros((B, 1), dtype=jnp.int32)
        for c in range(nch):
            v = x_ref[:, c * _CHUNK:(c + 1) * _CHUNK]
            col = jax.lax.broadcasted_iota(jnp.int32, (B, _CHUNK), 1) + c * _CHUNK
            for j in range(k):
                v = jnp.where(col == sels[j], -jnp.inf, v)
            cm = jnp.max(v, axis=1, keepdims=True)
            ci = jnp.min(jnp.where(v == cm, col, D), axis=1, keepdims=True)
            upd = cm > best_v
            best_v = jnp.where(upd, cm, best_v)
            best_i = jnp.where(upd, ci, best_i)
        sels.append(best_i)
    idx_ref[...] = jnp.concatenate(sels, axis=1)


def _sc_fill_body(D, ppw, idx_hbm, zsrc_hbm, out_hbm, zbuf, idxbuf, obuf, sem_z, sem_o):
    wid = lax.axis_index("s") * _NC + lax.axis_index("c")
    base_plane = wid * ppw

    pltpu.sync_copy(zsrc_hbm, zbuf)

    handles = []
    for j in range(ppw):
        p = base_plane + j
        dst = out_hbm.at[p // _K, p % _K]
        handles.append(pltpu.async_copy(zbuf, dst, sem_z))

    abase = (base_plane // 8) * 8
    pltpu.sync_copy(idx_hbm.at[pl.ds(abase, 32)], idxbuf)
    off = base_plane - abase

    for h in handles:
        h.wait()

    iota = lax.iota(jnp.int32, 16)
    c0 = idxbuf[pl.ds(0, 16)]
    c1 = idxbuf[pl.ds(16, 16)]
    ones_handles = []
    for j in range(ppw):
        l = off + j
        pos = jnp.maximum(
            jnp.max(jnp.where(iota == l, c0, -1)),
            jnp.max(jnp.where(iota == l - 16, c1, -1)),
        )
        seg = (pos // 16) * 16
        lane = pos - seg
        obuf[pl.ds(16 * j, 16)] = jnp.where(iota == lane, 1.0, 0.0).astype(jnp.float32)
        p = base_plane + j
        dst = out_hbm.at[p // _K, p % _K, pl.ds(seg, 16)]
        ones_handles.append(pltpu.async_copy(obuf.at[pl.ds(16 * j, 16)], dst, sem_o))
    for h in ones_handles:
        h.wait()


def kernel(similarities):
    B, D = similarities.shape
    idx = pl.pallas_call(
        _topk_body,
        out_shape=jax.ShapeDtypeStruct((B, _K), jnp.int32),
    )(similarities)

    nplanes = B * _K
    ppw = nplanes // _NW
    idx_pad = jnp.concatenate(
        [idx.reshape(-1), jnp.zeros((64,), jnp.int32)])
    zsrc = jnp.zeros((D,), jnp.float32)

    sc_fill = functools.partial(
        pl.kernel,
        out_type=jax.ShapeDtypeStruct((B, _K, D), jnp.float32),
        mesh=plsc.VectorSubcoreMesh(core_axis_name="c", subcore_axis_name="s"),
        compiler_params=pltpu.CompilerParams(needs_layout_passes=False),
        scratch_types=[
            pltpu.VMEM((D,), jnp.float32),
            pltpu.VMEM((32,), jnp.int32),
            pltpu.VMEM((16 * ppw,), jnp.float32),
            pltpu.SemaphoreType.DMA,
            pltpu.SemaphoreType.DMA,
        ],
    )(functools.partial(_sc_fill_body, D, ppw))

    out = sc_fill(idx_pad, zsrc)
    return idx, out
